# baseline (speedup 1.0000x reference)
import functools

import jax
import jax.numpy as jnp
import numpy as np
from jax import lax
from jax.experimental import pallas as pl
from jax.experimental.pallas import tpu as pltpu
from jax.experimental.pallas import tpu_sc as plsc

N = 10000
E = 320000
H = 128
FT_OUT = 64
NG = 512

E_REAL = E + N
NC, NS = 2, 16
NW = NC * NS
C = 64
CHUNKS = 164
EPW = C * CHUNKS
E_PAD = EPW * NW
N_PAD = 10240
ROWS_PT = N_PAD // NS
NBLK = 10
BLK = N // NBLK

_PERM = np.concatenate(
    [32 * f2 + np.concatenate([np.arange(0, 32, 2), np.arange(1, 32, 2)])
     for f2 in range(4)]).astype(np.int32)


def _edge_body(xt, srcr, gdstr, sdstr, attr, acc_o, den_o,
               acc_sh, den_sh,
               srcb0, gdstb0, sdstb0, srcb1, gdstb1, sdstb1,
               srcb2, gdstb2, sdstb2, srcb3, gdstb3, sdstb3,
               bufL0, bufR0, bufW0, exb0, bufL1, bufR1, bufW1, exb1, attv,
               iS0, iS1, iS2, iS3, gS0, gS1, sS0, sS1):
    cid = lax.axis_index("c")
    sid = lax.axis_index("s")
    wid = sid * NC + cid
    z16 = jnp.zeros((16,), jnp.float32)
    srcb = (srcb0, srcb1, srcb2, srcb3)
    gdstb = (gdstb0, gdstb1, gdstb2, gdstb3)
    sdstb = (sdstb0, sdstb1, sdstb2, sdstb3)
    iS = (iS0, iS1, iS2, iS3)
    bufL = (bufL0, bufL1)
    bufR = (bufR0, bufR1)
    bufW = (bufW0, bufW1)
    exb = (exb0, exb1)
    gS = (gS0, gS1)
    sS = (sS0, sS1)

    @pl.loop(0, C)
    def _zero(r):
        for f in range(8):
            bufW0[r, pl.ds(16 * f, 16)] = z16
        exb0[r, pl.ds(0, 16)] = z16

    base_r = sid * ROWS_PT
    for k in range(ROWS_PT // C):
        sl = pl.ds(base_r + k * C, C)
        pltpu.sync_copy(bufW0, acc_sh.at[sl])
        pltpu.sync_copy(exb0, den_sh.at[sl])
    pltpu.sync_copy(attr, attv)
    plsc.subcore_barrier()

    attregs = [attv[pl.ds(16 * f, 16)] for f in range(8)]
    iota16 = lax.iota(jnp.int32, 16)
    perms = {k: jnp.bitwise_xor(iota16, k) for k in (8, 4, 2, 1)}
    ebase = wid * EPW
    rbase = wid * CHUNKS

    def issue_idx(cidx, q):
        row = rbase + cidx
        pltpu.async_copy(srcr.at[row], srcb[q], iS[q])
        pltpu.async_copy(gdstr.at[row], gdstb[q], iS[q])
        pltpu.async_copy(sdstr.at[row], sdstb[q], iS[q])

    def wait_idx(cidx, q):
        row = rbase + cidx
        pltpu.make_async_copy(srcr.at[row], srcb[q], iS[q]).wait()
        pltpu.make_async_copy(gdstr.at[row], gdstb[q], iS[q]).wait()
        pltpu.make_async_copy(sdstr.at[row], sdstb[q], iS[q]).wait()

    CH = C // 2

    def issue_gather(q, b):
        for h0 in (0, CH):
            sl = pl.ds(h0, CH)
            pltpu.async_copy(xt.at[srcb[q].at[sl]], bufL[b].at[sl], gS[b])
            pltpu.async_copy(xt.at[gdstb[q].at[sl]], bufR[b].at[sl], gS[b])

    def wait_gather(q, b):
        for h0 in (0, CH):
            sl = pl.ds(h0, CH)
            pltpu.make_async_copy(xt.at[srcb[q].at[sl]], bufL[b].at[sl],
                                  gS[b]).wait()
            pltpu.make_async_copy(xt.at[gdstb[q].at[sl]], bufR[b].at[sl],
                                  gS[b]).wait()

    def issue_scatter(q, b):
        pltpu.async_copy(bufW[b], acc_sh.at[sdstb[q]], sS[b], add=True)
        pltpu.async_copy(exb[b], den_sh.at[sdstb[q]], sS[b], add=True)

    def wait_scatter(q, b):
        pltpu.make_async_copy(bufW[b], acc_sh.at[sdstb[q]], sS[b]).wait()
        pltpu.make_async_copy(exb[b], den_sh.at[sdstb[q]], sS[b]).wait()

    issue_idx(0, 0)
    issue_idx(1, 1)
    issue_idx(2, 2)
    wait_idx(0, 0)
    issue_gather(0, 0)

    @pl.loop(0, CHUNKS, step=4)
    def _quad(g):
        for k in range(4):
            cidx = g + k
            b = k % 2
            q = k
            base = ebase + cidx * C

            @pl.when(cidx > 0)
            def _():
                wait_scatter((q - 1) % 4, 1 - b)

            @pl.when(cidx + 1 < CHUNKS)
            def _():
                wait_idx(cidx + 1, (q + 1) % 4)
                issue_gather((q + 1) % 4, 1 - b)

            wait_gather(q, b)

            @pl.when(cidx + 3 < CHUNKS)
            def _():
                issue_idx(cidx + 3, (q + 3) % 4)

            mybufL, mybufR, mybufW, myexb = bufL[b], bufR[b], bufW[b], exb[b]

            @pl.loop(0, C, unroll=4)
            def _edge(c):
                lr_regs = []
                ps = z16
                for f2 in range(4):
                    lw = mybufL[c, pl.ds(16 * f2, 16)]
                    rw = mybufR[c, pl.ds(16 * f2, 16)]
                    le = lax.bitcast_convert_type(lw << 16, jnp.float32)
                    lo = lax.bitcast_convert_type(lw & np.int32(-65536),
                                                  jnp.float32)
                    re = lax.bitcast_convert_type(rw << 16, jnp.float32)
                    ro = lax.bitcast_convert_type(rw & np.int32(-65536),
                                                  jnp.float32)
                    ue = le + re
                    uo = lo + ro
                    ps = ps + jnp.maximum(ue, 0.2 * ue) * attregs[2 * f2]
                    ps = ps + jnp.maximum(uo, 0.2 * uo) * attregs[2 * f2 + 1]
                    lr_regs.append(le)
                    lr_regs.append(lo)
                for k2 in (8, 4, 2, 1):
                    ps = ps + ps.at[perms[k2]].get(mode="promise_in_bounds")
                scale = jnp.where(base + c < E_REAL, 1.0, 0.0)
                ex = jnp.exp(ps) * scale
                myexb[c, pl.ds(0, 16)] = jnp.where(iota16 == 0, ex, z16)
                for f in range(8):
                    mybufW[c, pl.ds(16 * f, 16)] = lr_regs[f] * ex

            issue_scatter(q, b)

    wait_scatter((CHUNKS - 1) % 4, 1)
    plsc.subcore_barrier()
    for k in range(ROWS_PT // C):
        sl = pl.ds(base_r + k * C, C)
        pltpu.sync_copy(acc_sh.at[sl], acc_o.at[cid, sl])
        pltpu.sync_copy(den_sh.at[sl], den_o.at[cid, sl])


_edge_sc = pl.kernel(
    _edge_body,
    out_type=[
        jax.ShapeDtypeStruct((NC, N_PAD, H), jnp.float32),
        jax.ShapeDtypeStruct((NC, N_PAD, 16), jnp.float32),
    ],
    mesh=plsc.VectorSubcoreMesh(core_axis_name="c", subcore_axis_name="s",
                                num_cores=NC, num_subcores=NS),
    scratch_types=(
        [
            pltpu.VMEM_SHARED((N_PAD, H), jnp.float32),
            pltpu.VMEM_SHARED((N_PAD, 16), jnp.float32),
        ]
        + 12 * [pltpu.VMEM((C,), jnp.int32)]
        + 2 * [
            pltpu.VMEM((C, H // 2), jnp.int32),
            pltpu.VMEM((C, H // 2), jnp.int32),
            pltpu.VMEM((C, H), jnp.float32),
            pltpu.VMEM((C, 16), jnp.float32),
        ]
        + [pltpu.VMEM((H,), jnp.float32)]
        + 8 * [pltpu.SemaphoreType.DMA]
    ),
    compiler_params=pltpu.CompilerParams(use_tc_tiling_on_sc=False),
)


def _pack_rows(y):
    return lax.bitcast_convert_type(y.reshape(y.shape[0], H // 2, 2),
                                    jnp.int32)


def _mm2_body(x_ref, wl_ref, bl_ref, wr_ref, br_ref, xl_ref, xr_ref):
    xb = x_ref[...]
    xl_ref[...] = (jnp.dot(xb, wl_ref[...], preferred_element_type=jnp.float32)
                   + bl_ref[...]).astype(jnp.bfloat16)
    xr_ref[...] = (jnp.dot(xb, wr_ref[...], preferred_element_type=jnp.float32)
                   + br_ref[...]).astype(jnp.bfloat16)


_mm2 = pl.pallas_call(
    _mm2_body,
    grid=(NBLK,),
    in_specs=[
        pl.BlockSpec((BLK, H), lambda i: (i, 0)),
        pl.BlockSpec((H, H), lambda i: (0, 0)),
        pl.BlockSpec((H,), lambda i: (0,)),
        pl.BlockSpec((H, H), lambda i: (0, 0)),
        pl.BlockSpec((H,), lambda i: (0,)),
    ],
    out_specs=[
        pl.BlockSpec((BLK, H), lambda i: (i, 0)),
        pl.BlockSpec((BLK, H), lambda i: (i, 0)),
    ],
    out_shape=[
        jax.ShapeDtypeStruct((N, H), jnp.bfloat16),
        jax.ShapeDtypeStruct((N, H), jnp.bfloat16),
    ],
)


def _comb_mm_body(acc_ref, den_ref, bias_ref, w_ref, b_ref, out_ref, *,
                  apply_relu):
    a = acc_ref[0] + acc_ref[1]
    d = den_ref[0, :, 0:1] + den_ref[1, :, 0:1]
    h = a / jnp.maximum(d, 1e-16) + bias_ref[...]
    if apply_relu:
        h = jnp.maximum(h, 0.0)
    out_ref[...] = (jnp.dot(h, w_ref[...], preferred_element_type=jnp.float32)
                    + b_ref[...]).astype(jnp.bfloat16)


def _make_comb_mm(apply_relu):
    return pl.pallas_call(
        functools.partial(_comb_mm_body, apply_relu=apply_relu),
        grid=(NBLK,),
        in_specs=[
            pl.BlockSpec((NC, BLK, H), lambda i: (0, i, 0)),
            pl.BlockSpec((NC, BLK, 16), lambda i: (0, i, 0)),
            pl.BlockSpec((H,), lambda i: (0,)),
            pl.BlockSpec((H, H), lambda i: (0, 0)),
            pl.BlockSpec((H,), lambda i: (0,)),
        ],
        out_specs=pl.BlockSpec((BLK, H), lambda i: (i, 0)),
        out_shape=jax.ShapeDtypeStruct((N, H), jnp.bfloat16),
    )


_comb_mm_relu = _make_comb_mm(True)


def _pool_body(acc_ref, den_ref, bias_ref, batch_ref, wlin_ref, blin_ref,
               out_ref, psum_ref, csum_ref):
    i = pl.program_id(0)

    @pl.when(i == 0)
    def _():
        psum_ref[...] = jnp.zeros((NG, H), jnp.float32)
        csum_ref[...] = jnp.zeros((NG, H), jnp.float32)

    a = acc_ref[0] + acc_ref[1]
    d = den_ref[0, :, 0:1] + den_ref[1, :, 0:1]
    h = a / jnp.maximum(d, 1e-16) + bias_ref[...]
    b = batch_ref[0, 0, :]
    gid = lax.broadcasted_iota(jnp.int32, (NG, BLK), 0)
    m = (gid == b[None, :]).astype(jnp.float32)
    psum_ref[...] += jnp.dot(m, h, preferred_element_type=jnp.float32)
    csum_ref[...] += jnp.dot(m, jnp.ones((BLK, H), jnp.float32),
                             preferred_element_type=jnp.float32)

    @pl.when(i == NBLK - 1)
    def _():
        pooled = psum_ref[...] / jnp.maximum(csum_ref[...], 1.0)
        out_ref[...] = jnp.dot(pooled, wlin_ref[...],
                               preferred_element_type=jnp.float32) + blin_ref[...]


_pool = pl.pallas_call(
    _pool_body,
    grid=(NBLK,),
    in_specs=[
        pl.BlockSpec((NC, BLK, H), lambda i: (0, i, 0)),
        pl.BlockSpec((NC, BLK, 16), lambda i: (0, i, 0)),
        pl.BlockSpec((H,), lambda i: (0,)),
        pl.BlockSpec((1, 1, BLK), lambda i: (i, 0, 0)),
        pl.BlockSpec((H, FT_OUT), lambda i: (0, 0)),
        pl.BlockSpec((FT_OUT,), lambda i: (0,)),
    ],
    out_specs=pl.BlockSpec((NG, FT_OUT), lambda i: (0, 0)),
    out_shape=jax.ShapeDtypeStruct((NG, FT_OUT), jnp.float32),
    scratch_shapes=[
        pltpu.VMEM((NG, H), jnp.float32),
        pltpu.VMEM((NG, H), jnp.float32),
    ],
)


def kernel(x, edge_index, batch, Wl1, bl1, Wr1, br1, att1, bias1,
           W2, b2, att2, bias2, W3, b3, att3, bias3, Wlin, blin):
    loop = jnp.arange(N, dtype=edge_index.dtype)
    pad = jnp.zeros((E_PAD - E_REAL,), edge_index.dtype)
    src = jnp.concatenate([edge_index[0], loop, pad]).reshape(-1, C)
    dst = jnp.concatenate([edge_index[1], loop, pad]).reshape(-1, C)
    dst1 = dst + N

    perm = jnp.asarray(_PERM)
    xl1, xr1 = _mm2(x, Wl1, bl1, Wr1, br1)
    xcat = _pack_rows(jnp.concatenate([xl1, xr1], axis=0))
    acc1, den1 = _edge_sc(xcat, src, dst1, dst, att1[perm])
    xl2 = _comb_mm_relu(acc1, den1, bias1[perm], W2[perm, :], b2)
    acc2, den2 = _edge_sc(_pack_rows(xl2), src, dst, dst, att2[perm])
    xl3 = _comb_mm_relu(acc2, den2, bias2[perm], W3[perm, :], b3)
    acc3, den3 = _edge_sc(_pack_rows(xl3), src, dst, dst, att3[perm])
    return _pool(acc3, den3, bias3[perm], batch.reshape(NBLK, 1, BLK),
                 Wlin[perm, :], blin)


# speedup vs baseline: 1.0887x; 1.0887x over previous
import functools

import jax
import jax.numpy as jnp
import numpy as np
from jax import lax
from jax.experimental import pallas as pl
from jax.experimental.pallas import tpu as pltpu
from jax.experimental.pallas import tpu_sc as plsc

N = 10000
E = 320000
H = 128
FT_OUT = 64
NG = 512

E_REAL = E + N
NC, NS = 2, 16
NW = NC * NS
C = 64
CHUNKS = 164
EPW = C * CHUNKS
E_PAD = EPW * NW
N_PAD = 10240
ROWS_PT = N_PAD // NS
NBLK = 10
BLK = N // NBLK

_PERM = np.concatenate(
    [32 * f2 + np.concatenate([np.arange(0, 32, 2), np.arange(1, 32, 2)])
     for f2 in range(4)]).astype(np.int32)


def _edge_body(xt, idxr, attr, acc_o,
               acc_sh,
               idx0, idx1, idx2, idx3,
               bufL0, bufR0, bufW0, sdst0, bufL1, bufR1, bufW1, sdst1, attv,
               iS0, iS1, iS2, iS3, gS0, gS1, sS0, sS1):
    cid = lax.axis_index("c")
    sid = lax.axis_index("s")
    wid = sid * NC + cid
    z16 = jnp.zeros((16,), jnp.float32)
    idxb = (idx0, idx1, idx2, idx3)
    iS = (iS0, iS1, iS2, iS3)
    bufL = (bufL0, bufL1)
    bufR = (bufR0, bufR1)
    bufW = (bufW0, bufW1)
    sdst = (sdst0, sdst1)
    gS = (gS0, gS1)
    sS = (sS0, sS1)
    W = H + 16

    @pl.loop(0, C)
    def _zero(r):
        for f in range(W // 16):
            bufW0[r, pl.ds(16 * f, 16)] = z16

    base_r = sid * ROWS_PT
    for k in range(ROWS_PT // C):
        pltpu.sync_copy(bufW0, acc_sh.at[pl.ds(base_r + k * C, C)])
    pltpu.sync_copy(attr, attv)
    plsc.subcore_barrier()

    attregs = [attv[pl.ds(16 * f, 16)] for f in range(8)]
    iota16 = lax.iota(jnp.int32, 16)
    perms = {k: jnp.bitwise_xor(iota16, k) for k in (8, 4, 2, 1)}
    ebase = wid * EPW
    rbase = wid * CHUNKS

    def issue_idx(cidx, q):
        pltpu.async_copy(idxr.at[rbase + cidx], idxb[q], iS[q])

    def wait_idx(cidx, q):
        pltpu.make_async_copy(idxr.at[rbase + cidx], idxb[q], iS[q]).wait()

    def issue_gather(q, b):
        pltpu.async_copy(xt.at[idxb[q].at[0]], bufL[b], gS[b])
        pltpu.async_copy(xt.at[idxb[q].at[1]], bufR[b], gS[b])

    def wait_gather(q, b):
        pltpu.make_async_copy(xt.at[idxb[q].at[0]], bufL[b], gS[b]).wait()
        pltpu.make_async_copy(xt.at[idxb[q].at[1]], bufR[b], gS[b]).wait()

    def issue_scatter(b):
        pltpu.async_copy(bufW[b], acc_sh.at[sdst[b]], sS[b], add=True)

    def wait_scatter(b):
        pltpu.make_async_copy(bufW[b], acc_sh.at[sdst[b]], sS[b]).wait()

    issue_idx(0, 0)
    issue_idx(1, 1)
    issue_idx(2, 2)
    wait_idx(0, 0)
    issue_gather(0, 0)

    @pl.loop(0, CHUNKS, step=4)
    def _quad(g):
        for k in range(4):
            cidx = g + k
            b = k % 2
            q = k
            base = ebase + cidx * C

            @pl.when(cidx + 1 < CHUNKS)
            def _():
                wait_idx(cidx + 1, (q + 1) % 4)
                issue_gather((q + 1) % 4, 1 - b)

            wait_gather(q, b)

            @pl.when(cidx + 3 < CHUNKS)
            def _():
                issue_idx(cidx + 3, (q + 3) % 4)

            @pl.when(cidx > 1)
            def _():
                wait_scatter(b)

            mybufL, mybufR, mybufW = bufL[b], bufR[b], bufW[b]
            myidx, mysdst = idxb[q], sdst[b]

            for k3 in range(C // 16):
                mysdst[pl.ds(16 * k3, 16)] = myidx[2, pl.ds(16 * k3, 16)]

            @pl.loop(0, C, unroll=4)
            def _edge(c):
                lr_regs = []
                ps = z16
                for f2 in range(4):
                    lw = mybufL[c, pl.ds(16 * f2, 16)]
                    rw = mybufR[c, pl.ds(16 * f2, 16)]
                    le = lax.bitcast_convert_type(lw << 16, jnp.float32)
                    lo = lax.bitcast_convert_type(lw & np.int32(-65536),
                                                  jnp.float32)
                    re = lax.bitcast_convert_type(rw << 16, jnp.float32)
                    ro = lax.bitcast_convert_type(rw & np.int32(-65536),
                                                  jnp.float32)
                    ue = le + re
                    uo = lo + ro
                    ps = ps + jnp.maximum(ue, 0.2 * ue) * attregs[2 * f2]
                    ps = ps + jnp.maximum(uo, 0.2 * uo) * attregs[2 * f2 + 1]
                    lr_regs.append(le)
                    lr_regs.append(lo)
                for k2 in (8, 4, 2, 1):
                    ps = ps + ps.at[perms[k2]].get(mode="promise_in_bounds")
                scale = jnp.where(base + c < E_REAL, 1.0, 0.0)
                ex = jnp.exp(ps) * scale
                for f in range(8):
                    mybufW[c, pl.ds(16 * f, 16)] = lr_regs[f] * ex
                mybufW[c, pl.ds(H, 16)] = jnp.where(iota16 == 0, ex, z16)

            issue_scatter(b)

    wait_scatter(0)
    wait_scatter(1)
    plsc.subcore_barrier()
    for k in range(ROWS_PT // C):
        sl = pl.ds(base_r + k * C, C)
        pltpu.sync_copy(acc_sh.at[sl], acc_o.at[cid, sl])


_edge_sc = pl.kernel(
    _edge_body,
    out_type=jax.ShapeDtypeStruct((NC, N_PAD, H + 16), jnp.float32),
    mesh=plsc.VectorSubcoreMesh(core_axis_name="c", subcore_axis_name="s",
                                num_cores=NC, num_subcores=NS),
    scratch_types=(
        [pltpu.VMEM_SHARED((N_PAD, H + 16), jnp.float32)]
        + 4 * [pltpu.VMEM((3, C), jnp.int32)]
        + 2 * [
            pltpu.VMEM((C, H // 2), jnp.int32),
            pltpu.VMEM((C, H // 2), jnp.int32),
            pltpu.VMEM((C, H + 16), jnp.float32),
            pltpu.VMEM((C,), jnp.int32),
        ]
        + [pltpu.VMEM((H,), jnp.float32)]
        + 8 * [pltpu.SemaphoreType.DMA]
    ),
    compiler_params=pltpu.CompilerParams(use_tc_tiling_on_sc=False),
)


def _pack_rows(y):
    return lax.bitcast_convert_type(y.reshape(y.shape[0], H // 2, 2),
                                    jnp.int32)


def _mm2_body(x_ref, wl_ref, bl_ref, wr_ref, br_ref, xl_ref, xr_ref):
    xb = x_ref[...]
    xl_ref[...] = (jnp.dot(xb, wl_ref[...], preferred_element_type=jnp.float32)
                   + bl_ref[...]).astype(jnp.bfloat16)
    xr_ref[...] = (jnp.dot(xb, wr_ref[...], preferred_element_type=jnp.float32)
                   + br_ref[...]).astype(jnp.bfloat16)


_mm2 = pl.pallas_call(
    _mm2_body,
    grid=(NBLK,),
    in_specs=[
        pl.BlockSpec((BLK, H), lambda i: (i, 0)),
        pl.BlockSpec((H, H), lambda i: (0, 0)),
        pl.BlockSpec((H,), lambda i: (0,)),
        pl.BlockSpec((H, H), lambda i: (0, 0)),
        pl.BlockSpec((H,), lambda i: (0,)),
    ],
    out_specs=[
        pl.BlockSpec((BLK, H), lambda i: (i, 0)),
        pl.BlockSpec((BLK, H), lambda i: (i, 0)),
    ],
    out_shape=[
        jax.ShapeDtypeStruct((N, H), jnp.bfloat16),
        jax.ShapeDtypeStruct((N, H), jnp.bfloat16),
    ],
)


def _comb_mm_body(acc_ref, bias_ref, w_ref, b_ref, out_ref, *,
                  apply_relu):
    a = acc_ref[0, :, :H] + acc_ref[1, :, :H]
    d = acc_ref[0, :, H:H + 1] + acc_ref[1, :, H:H + 1]
    h = a / jnp.maximum(d, 1e-16) + bias_ref[...]
    if apply_relu:
        h = jnp.maximum(h, 0.0)
    out_ref[...] = (jnp.dot(h, w_ref[...], preferred_element_type=jnp.float32)
                    + b_ref[...]).astype(jnp.bfloat16)


def _make_comb_mm(apply_relu):
    return pl.pallas_call(
        functools.partial(_comb_mm_body, apply_relu=apply_relu),
        grid=(NBLK,),
        in_specs=[
            pl.BlockSpec((NC, BLK, H + 16), lambda i: (0, i, 0)),
            pl.BlockSpec((H,), lambda i: (0,)),
            pl.BlockSpec((H, H), lambda i: (0, 0)),
            pl.BlockSpec((H,), lambda i: (0,)),
        ],
        out_specs=pl.BlockSpec((BLK, H), lambda i: (i, 0)),
        out_shape=jax.ShapeDtypeStruct((N, H), jnp.bfloat16),
    )


_comb_mm_relu = _make_comb_mm(True)


def _pool_body(acc_ref, bias_ref, batch_ref, wlin_ref, blin_ref,
               out_ref, psum_ref, csum_ref):
    i = pl.program_id(0)

    @pl.when(i == 0)
    def _():
        psum_ref[...] = jnp.zeros((NG, H), jnp.float32)
        csum_ref[...] = jnp.zeros((NG, H), jnp.float32)

    a = acc_ref[0, :, :H] + acc_ref[1, :, :H]
    d = acc_ref[0, :, H:H + 1] + acc_ref[1, :, H:H + 1]
    h = a / jnp.maximum(d, 1e-16) + bias_ref[...]
    b = batch_ref[0, 0, :]
    gid = lax.broadcasted_iota(jnp.int32, (NG, BLK), 0)
    m = (gid == b[None, :]).astype(jnp.float32)
    psum_ref[...] += jnp.dot(m, h, preferred_element_type=jnp.float32)
    csum_ref[...] += jnp.dot(m, jnp.ones((BLK, H), jnp.float32),
                             preferred_element_type=jnp.float32)

    @pl.when(i == NBLK - 1)
    def _():
        pooled = psum_ref[...] / jnp.maximum(csum_ref[...], 1.0)
        out_ref[...] = jnp.dot(pooled, wlin_ref[...],
                               preferred_element_type=jnp.float32) + blin_ref[...]


_pool = pl.pallas_call(
    _pool_body,
    grid=(NBLK,),
    in_specs=[
        pl.BlockSpec((NC, BLK, H + 16), lambda i: (0, i, 0)),
        pl.BlockSpec((H,), lambda i: (0,)),
        pl.BlockSpec((1, 1, BLK), lambda i: (i, 0, 0)),
        pl.BlockSpec((H, FT_OUT), lambda i: (0, 0)),
        pl.BlockSpec((FT_OUT,), lambda i: (0,)),
    ],
    out_specs=pl.BlockSpec((NG, FT_OUT), lambda i: (0, 0)),
    out_shape=jax.ShapeDtypeStruct((NG, FT_OUT), jnp.float32),
    scratch_shapes=[
        pltpu.VMEM((NG, H), jnp.float32),
        pltpu.VMEM((NG, H), jnp.float32),
    ],
)


def kernel(x, edge_index, batch, Wl1, bl1, Wr1, br1, att1, bias1,
           W2, b2, att2, bias2, W3, b3, att3, bias3, Wlin, blin):
    loop = jnp.arange(N, dtype=edge_index.dtype)
    pad = jnp.zeros((E_PAD - E_REAL,), edge_index.dtype)
    src = jnp.concatenate([edge_index[0], loop, pad]).reshape(-1, 1, C)
    dst = jnp.concatenate([edge_index[1], loop, pad]).reshape(-1, 1, C)
    idx1 = jnp.concatenate([src, dst + N, dst], axis=1)
    idx23 = jnp.concatenate([src, dst, dst], axis=1)

    perm = jnp.asarray(_PERM)
    xl1, xr1 = _mm2(x, Wl1, bl1, Wr1, br1)
    xcat = _pack_rows(jnp.concatenate([xl1, xr1], axis=0))
    acc1 = _edge_sc(xcat, idx1, att1[perm])
    xl2 = _comb_mm_relu(acc1, bias1[perm], W2[perm, :], b2)
    acc2 = _edge_sc(_pack_rows(xl2), idx23, att2[perm])
    xl3 = _comb_mm_relu(acc2, bias2[perm], W3[perm, :], b3)
    acc3 = _edge_sc(_pack_rows(xl3), idx23, att3[perm])
    return _pool(acc3, bias3[perm], batch.reshape(NBLK, 1, BLK),
                 Wlin[perm, :], blin)


# speedup vs baseline: 1.0896x; 1.0008x over previous
import functools

import jax
import jax.numpy as jnp
import numpy as np
from jax import lax
from jax.experimental import pallas as pl
from jax.experimental.pallas import tpu as pltpu
from jax.experimental.pallas import tpu_sc as plsc

N = 10000
E = 320000
H = 128
FT_OUT = 64
NG = 512

E_REAL = E + N
NC, NS = 2, 16
NW = NC * NS
C = 64
CHUNKS = 164
EPW = C * CHUNKS
E_PAD = EPW * NW
N_PAD = 10240
ROWS_PT = N_PAD // NS
NBLK = 10
BLK = N // NBLK

_PERM = np.concatenate(
    [32 * f2 + np.concatenate([np.arange(0, 32, 2), np.arange(1, 32, 2)])
     for f2 in range(4)]).astype(np.int32)


def _edge_body(xt, idxr, attr, acc_o,
               acc_sh,
               idx0, idx1, idx2, idx3,
               bufL0, bufR0, bufW0, sdst0, bufL1, bufR1, bufW1, sdst1, attv,
               iS0, iS1, iS2, iS3, gS0, gS1, sS0, sS1):
    cid = lax.axis_index("c")
    sid = lax.axis_index("s")
    wid = sid * NC + cid
    z16 = jnp.zeros((16,), jnp.float32)
    idxb = (idx0, idx1, idx2, idx3)
    iS = (iS0, iS1, iS2, iS3)
    bufL = (bufL0, bufL1)
    bufR = (bufR0, bufR1)
    bufW = (bufW0, bufW1)
    sdst = (sdst0, sdst1)
    gS = (gS0, gS1)
    sS = (sS0, sS1)
    W = H + 16

    @pl.loop(0, C)
    def _zero(r):
        for f in range(W // 16):
            bufW0[r, pl.ds(16 * f, 16)] = z16

    base_r = sid * ROWS_PT
    for k in range(ROWS_PT // C):
        pltpu.sync_copy(bufW0, acc_sh.at[pl.ds(base_r + k * C, C)])
    pltpu.sync_copy(attr, attv)
    plsc.subcore_barrier()

    attregs = [attv[pl.ds(16 * f, 16)] for f in range(8)]
    iota16 = lax.iota(jnp.int32, 16)
    perms = {k: jnp.bitwise_xor(iota16, k) for k in (8, 4, 2, 1)}
    ebase = wid * EPW
    rbase = wid * CHUNKS

    def issue_idx(cidx, q):
        pltpu.async_copy(idxr.at[rbase + cidx], idxb[q], iS[q])

    def wait_idx(cidx, q):
        pltpu.make_async_copy(idxr.at[rbase + cidx], idxb[q], iS[q]).wait()

    def issue_gather(q, b):
        pltpu.async_copy(xt.at[idxb[q].at[0]], bufL[b], gS[b])
        pltpu.async_copy(xt.at[idxb[q].at[1]], bufR[b], gS[b])

    def wait_gather(q, b):
        pltpu.make_async_copy(xt.at[idxb[q].at[0]], bufL[b], gS[b]).wait()
        pltpu.make_async_copy(xt.at[idxb[q].at[1]], bufR[b], gS[b]).wait()

    def issue_scatter(b):
        pltpu.async_copy(bufW[b], acc_sh.at[sdst[b]], sS[b], add=True)

    def wait_scatter(b):
        pltpu.make_async_copy(bufW[b], acc_sh.at[sdst[b]], sS[b]).wait()

    issue_idx(0, 0)
    issue_idx(1, 1)
    issue_idx(2, 2)
    wait_idx(0, 0)
    issue_gather(0, 0)

    @pl.loop(0, CHUNKS, step=4)
    def _quad(g):
        for k in range(4):
            cidx = g + k
            b = k % 2
            q = k
            base = ebase + cidx * C

            @pl.when(cidx + 1 < CHUNKS)
            def _():
                wait_idx(cidx + 1, (q + 1) % 4)
                issue_gather((q + 1) % 4, 1 - b)

            @pl.when(cidx + 3 < CHUNKS)
            def _():
                issue_idx(cidx + 3, (q + 3) % 4)

            @pl.when(cidx > 1)
            def _():
                wait_scatter(b)

            mybufL, mybufR, mybufW = bufL[b], bufR[b], bufW[b]
            myidx, mysdst = idxb[q], sdst[b]

            for k3 in range(C // 16):
                mysdst[pl.ds(16 * k3, 16)] = myidx[2, pl.ds(16 * k3, 16)]

            wait_gather(q, b)

            @pl.loop(0, C, unroll=4)
            def _edge(c):
                lr_regs = []
                ps = z16
                for f2 in range(4):
                    lw = mybufL[c, pl.ds(16 * f2, 16)]
                    rw = mybufR[c, pl.ds(16 * f2, 16)]
                    le = lax.bitcast_convert_type(lw << 16, jnp.float32)
                    lo = lax.bitcast_convert_type(lw & np.int32(-65536),
                                                  jnp.float32)
                    re = lax.bitcast_convert_type(rw << 16, jnp.float32)
                    ro = lax.bitcast_convert_type(rw & np.int32(-65536),
                                                  jnp.float32)
                    ue = le + re
                    uo = lo + ro
                    ps = ps + jnp.maximum(ue, 0.2 * ue) * attregs[2 * f2]
                    ps = ps + jnp.maximum(uo, 0.2 * uo) * attregs[2 * f2 + 1]
                    lr_regs.append(le)
                    lr_regs.append(lo)
                for k2 in (8, 4, 2, 1):
                    ps = ps + ps.at[perms[k2]].get(mode="promise_in_bounds")
                scale = jnp.where(base + c < E_REAL, 1.0, 0.0)
                ex = jnp.exp(ps) * scale
                for f in range(8):
                    mybufW[c, pl.ds(16 * f, 16)] = lr_regs[f] * ex
                mybufW[c, pl.ds(H, 16)] = jnp.where(iota16 == 0, ex, z16)

            issue_scatter(b)

    wait_scatter(0)
    wait_scatter(1)
    plsc.subcore_barrier()
    for k in range(ROWS_PT // C):
        sl = pl.ds(base_r + k * C, C)
        pltpu.sync_copy(acc_sh.at[sl], acc_o.at[cid, sl])


_edge_sc = pl.kernel(
    _edge_body,
    out_type=jax.ShapeDtypeStruct((NC, N_PAD, H + 16), jnp.float32),
    mesh=plsc.VectorSubcoreMesh(core_axis_name="c", subcore_axis_name="s",
                                num_cores=NC, num_subcores=NS),
    scratch_types=(
        [pltpu.VMEM_SHARED((N_PAD, H + 16), jnp.float32)]
        + 4 * [pltpu.VMEM((3, C), jnp.int32)]
        + 2 * [
            pltpu.VMEM((C, H // 2), jnp.int32),
            pltpu.VMEM((C, H // 2), jnp.int32),
            pltpu.VMEM((C, H + 16), jnp.float32),
            pltpu.VMEM((C,), jnp.int32),
        ]
        + [pltpu.VMEM((H,), jnp.float32)]
        + 8 * [pltpu.SemaphoreType.DMA]
    ),
    compiler_params=pltpu.CompilerParams(use_tc_tiling_on_sc=False),
)


def _pack_rows(y):
    return lax.bitcast_convert_type(y.reshape(y.shape[0], H // 2, 2),
                                    jnp.int32)


def _mm2_body(x_ref, wl_ref, bl_ref, wr_ref, br_ref, xl_ref, xr_ref):
    xb = x_ref[...]
    xl_ref[...] = (jnp.dot(xb, wl_ref[...], preferred_element_type=jnp.float32)
                   + bl_ref[...]).astype(jnp.bfloat16)
    xr_ref[...] = (jnp.dot(xb, wr_ref[...], preferred_element_type=jnp.float32)
                   + br_ref[...]).astype(jnp.bfloat16)


_mm2 = pl.pallas_call(
    _mm2_body,
    grid=(NBLK,),
    in_specs=[
        pl.BlockSpec((BLK, H), lambda i: (i, 0)),
        pl.BlockSpec((H, H), lambda i: (0, 0)),
        pl.BlockSpec((H,), lambda i: (0,)),
        pl.BlockSpec((H, H), lambda i: (0, 0)),
        pl.BlockSpec((H,), lambda i: (0,)),
    ],
    out_specs=[
        pl.BlockSpec((BLK, H), lambda i: (i, 0)),
        pl.BlockSpec((BLK, H), lambda i: (i, 0)),
    ],
    out_shape=[
        jax.ShapeDtypeStruct((N, H), jnp.bfloat16),
        jax.ShapeDtypeStruct((N, H), jnp.bfloat16),
    ],
)


def _comb_mm_body(acc_ref, bias_ref, w_ref, b_ref, out_ref, *,
                  apply_relu):
    a = acc_ref[0, :, :H] + acc_ref[1, :, :H]
    d = acc_ref[0, :, H:H + 1] + acc_ref[1, :, H:H + 1]
    h = a / jnp.maximum(d, 1e-16) + bias_ref[...]
    if apply_relu:
        h = jnp.maximum(h, 0.0)
    out_ref[...] = (jnp.dot(h, w_ref[...], preferred_element_type=jnp.float32)
                    + b_ref[...]).astype(jnp.bfloat16)


def _make_comb_mm(apply_relu):
    return pl.pallas_call(
        functools.partial(_comb_mm_body, apply_relu=apply_relu),
        grid=(NBLK,),
        in_specs=[
            pl.BlockSpec((NC, BLK, H + 16), lambda i: (0, i, 0)),
            pl.BlockSpec((H,), lambda i: (0,)),
            pl.BlockSpec((H, H), lambda i: (0, 0)),
            pl.BlockSpec((H,), lambda i: (0,)),
        ],
        out_specs=pl.BlockSpec((BLK, H), lambda i: (i, 0)),
        out_shape=jax.ShapeDtypeStruct((N, H), jnp.bfloat16),
    )


_comb_mm_relu = _make_comb_mm(True)


def _pool_body(acc_ref, bias_ref, batch_ref, wlin_ref, blin_ref,
               out_ref, psum_ref, csum_ref):
    i = pl.program_id(0)

    @pl.when(i == 0)
    def _():
        psum_ref[...] = jnp.zeros((NG, H), jnp.float32)
        csum_ref[...] = jnp.zeros((NG, H), jnp.float32)

    a = acc_ref[0, :, :H] + acc_ref[1, :, :H]
    d = acc_ref[0, :, H:H + 1] + acc_ref[1, :, H:H + 1]
    h = a / jnp.maximum(d, 1e-16) + bias_ref[...]
    b = batch_ref[0, 0, :]
    gid = lax.broadcasted_iota(jnp.int32, (NG, BLK), 0)
    m = (gid == b[None, :]).astype(jnp.float32)
    psum_ref[...] += jnp.dot(m, h, preferred_element_type=jnp.float32)
    csum_ref[...] += jnp.dot(m, jnp.ones((BLK, H), jnp.float32),
                             preferred_element_type=jnp.float32)

    @pl.when(i == NBLK - 1)
    def _():
        pooled = psum_ref[...] / jnp.maximum(csum_ref[...], 1.0)
        out_ref[...] = jnp.dot(pooled, wlin_ref[...],
                               preferred_element_type=jnp.float32) + blin_ref[...]


_pool = pl.pallas_call(
    _pool_body,
    grid=(NBLK,),
    in_specs=[
        pl.BlockSpec((NC, BLK, H + 16), lambda i: (0, i, 0)),
        pl.BlockSpec((H,), lambda i: (0,)),
        pl.BlockSpec((1, 1, BLK), lambda i: (i, 0, 0)),
        pl.BlockSpec((H, FT_OUT), lambda i: (0, 0)),
        pl.BlockSpec((FT_OUT,), lambda i: (0,)),
    ],
    out_specs=pl.BlockSpec((NG, FT_OUT), lambda i: (0, 0)),
    out_shape=jax.ShapeDtypeStruct((NG, FT_OUT), jnp.float32),
    scratch_shapes=[
        pltpu.VMEM((NG, H), jnp.float32),
        pltpu.VMEM((NG, H), jnp.float32),
    ],
)


def kernel(x, edge_index, batch, Wl1, bl1, Wr1, br1, att1, bias1,
           W2, b2, att2, bias2, W3, b3, att3, bias3, Wlin, blin):
    loop = jnp.arange(N, dtype=edge_index.dtype)
    pad = jnp.zeros((E_PAD - E_REAL,), edge_index.dtype)
    src = jnp.concatenate([edge_index[0], loop, pad]).reshape(-1, 1, C)
    dst = jnp.concatenate([edge_index[1], loop, pad]).reshape(-1, 1, C)
    idx1 = jnp.concatenate([src, dst + N, dst], axis=1)
    idx23 = jnp.concatenate([src, dst, dst], axis=1)

    perm = jnp.asarray(_PERM)
    xl1, xr1 = _mm2(x, Wl1, bl1, Wr1, br1)
    xcat = _pack_rows(jnp.concatenate([xl1, xr1], axis=0))
    acc1 = _edge_sc(xcat, idx1, att1[perm])
    xl2 = _comb_mm_relu(acc1, bias1[perm], W2[perm, :], b2)
    acc2 = _edge_sc(_pack_rows(xl2), idx23, att2[perm])
    xl3 = _comb_mm_relu(acc2, bias2[perm], W3[perm, :], b3)
    acc3 = _edge_sc(_pack_rows(xl3), idx23, att3[perm])
    return _pool(acc3, bias3[perm], batch.reshape(NBLK, 1, BLK),
                 Wlin[perm, :], blin)
